# baseline (speedup 1.0000x reference)
import functools

import jax
import jax.numpy as jnp
from jax import lax
from jax.experimental import pallas as pl
from jax.experimental.pallas import tpu as pltpu
from jax.experimental.pallas import tpu_sc as plsc

_NUM_CORES = 2
_NUM_SUBCORES = 16
_NUM_WORKERS = _NUM_CORES * _NUM_SUBCORES


def kernel(tokens, W_pos):
    batch = tokens.shape[0]
    seq_len = tokens.shape[1]
    d_model = W_pos.shape[1]
    rows_per_worker = seq_len // _NUM_WORKERS

    mesh = plsc.VectorSubcoreMesh(core_axis_name="c", subcore_axis_name="s")

    @functools.partial(
        pl.kernel,
        mesh=mesh,
        out_type=jax.ShapeDtypeStruct((batch, seq_len, d_model), W_pos.dtype),
        scratch_types=[
            pltpu.VMEM((rows_per_worker, d_model), W_pos.dtype),
            pltpu.SemaphoreType.DMA,
            pltpu.SemaphoreType.DMA,
        ],
    )
    def sc_bcast(w_hbm, out_hbm, rows_v, in_sem, out_sem):
        wid = lax.axis_index("s") * _NUM_CORES + lax.axis_index("c")
        base = wid * rows_per_worker

        load = pltpu.make_async_copy(
            w_hbm.at[pl.ds(base, rows_per_worker)], rows_v, in_sem
        )
        load.start()
        load.wait()

        stores = []
        for j in range(batch):
            st = pltpu.make_async_copy(
                rows_v, out_hbm.at[j, pl.ds(base, rows_per_worker)], out_sem
            )
            st.start()
            stores.append(st)
        for st in stores:
            st.wait()

    return sc_bcast(W_pos[:seq_len])


# speedup vs baseline: 2.7512x; 2.7512x over previous
import jax
import jax.numpy as jnp
from jax.experimental import pallas as pl
from jax.experimental.pallas import tpu as pltpu

_N_CHUNKS = 8


def _bcast_kernel(w_hbm, out_hbm, w_vmem, in_sems, out_sems):
    batch = out_hbm.shape[0]
    seq_len = w_hbm.shape[0]
    chunk = seq_len // _N_CHUNKS

    loads = [
        pltpu.make_async_copy(
            w_hbm.at[pl.ds(i * chunk, chunk)],
            w_vmem.at[pl.ds(i * chunk, chunk)],
            in_sems.at[i],
        )
        for i in range(_N_CHUNKS)
    ]
    for ld in loads:
        ld.start()

    stores = []
    for i in range(_N_CHUNKS):
        loads[i].wait()
        for j in range(batch):
            st = pltpu.make_async_copy(
                w_vmem.at[pl.ds(i * chunk, chunk)],
                out_hbm.at[j, pl.ds(i * chunk, chunk)],
                out_sems.at[j],
            )
            st.start()
            stores.append(st)
    for st in stores:
        st.wait()


def kernel(tokens, W_pos):
    batch = tokens.shape[0]
    seq_len = tokens.shape[1]
    d_model = W_pos.shape[1]

    return pl.pallas_call(
        _bcast_kernel,
        in_specs=[pl.BlockSpec(memory_space=pl.ANY)],
        out_specs=pl.BlockSpec(memory_space=pl.ANY),
        out_shape=jax.ShapeDtypeStruct((batch, seq_len, d_model), W_pos.dtype),
        scratch_shapes=[
            pltpu.VMEM((seq_len, d_model), W_pos.dtype),
            pltpu.SemaphoreType.DMA((_N_CHUNKS,)),
            pltpu.SemaphoreType.DMA((batch,)),
        ],
    )(W_pos[:seq_len])


# speedup vs baseline: 2.7556x; 1.0016x over previous
import jax
import jax.numpy as jnp
from jax.experimental import pallas as pl
from jax.experimental.pallas import tpu as pltpu

_N_CHUNKS = 16


def _bcast_kernel(w_hbm, out_hbm, w_vmem, in_sems, out_sems):
    batch = out_hbm.shape[0]
    seq_len = w_hbm.shape[0]
    chunk = seq_len // _N_CHUNKS

    loads = [
        pltpu.make_async_copy(
            w_hbm.at[pl.ds(i * chunk, chunk)],
            w_vmem.at[pl.ds(i * chunk, chunk)],
            in_sems.at[i],
        )
        for i in range(_N_CHUNKS)
    ]
    for ld in loads:
        ld.start()

    stores = []
    for i in range(_N_CHUNKS):
        loads[i].wait()
        for j in range(batch):
            st = pltpu.make_async_copy(
                w_vmem.at[pl.ds(i * chunk, chunk)],
                out_hbm.at[j, pl.ds(i * chunk, chunk)],
                out_sems.at[j],
            )
            st.start()
            stores.append(st)
    for st in stores:
        st.wait()


def kernel(tokens, W_pos):
    batch = tokens.shape[0]
    seq_len = tokens.shape[1]
    d_model = W_pos.shape[1]

    return pl.pallas_call(
        _bcast_kernel,
        in_specs=[pl.BlockSpec(memory_space=pl.ANY)],
        out_specs=pl.BlockSpec(memory_space=pl.ANY),
        out_shape=jax.ShapeDtypeStruct((batch, seq_len, d_model), W_pos.dtype),
        scratch_shapes=[
            pltpu.VMEM((seq_len, d_model), W_pos.dtype),
            pltpu.SemaphoreType.DMA((_N_CHUNKS,)),
            pltpu.SemaphoreType.DMA((batch,)),
        ],
    )(W_pos[:seq_len])
